# baseline (speedup 1.0000x reference)
import functools

import jax
import jax.numpy as jnp
from jax import lax
from jax.experimental import pallas as pl
from jax.experimental.pallas import tpu as pltpu
from jax.experimental.pallas import tpu_sc as plsc

N = 10000
E = 320000
D = 128
NC = 2
NS = 16
NW = NC * NS
L = 16

EB = 128
ROWS_ALL = 2560
E_PAD = ROWS_ALL * EB
ROWS_PER_TILE_SC = ROWS_ALL // NS
ROWS_PER_W = ROWS_ALL // NW
NPAD = 10240
NODES_PER_TILE = NPAD // NS
NBUF = 2
NMB = 2
NIB = 2 * NBUF
NDS = 4

_mesh = plsc.VectorSubcoreMesh(
    core_axis_name="c", subcore_axis_name="s", num_cores=NC, num_subcores=NS)


def _lane_bcast(v, lane):
    idx = jnp.full((L, 1), lane, jnp.int32)
    dn = lax.GatherDimensionNumbers(
        offset_dims=(), collapsed_slice_dims=(0,), start_index_map=(0,))
    return lax.gather(v, idx, dn, slice_sizes=(1,),
                      mode=lax.GatherScatterMode.PROMISE_IN_BOUNDS)


@functools.partial(
    pl.kernel,
    out_type=jax.ShapeDtypeStruct((ROWS_ALL, EB), jnp.float32),
    mesh=_mesh,
    compiler_params=pltpu.CompilerParams(needs_layout_passes=False, use_tc_tiling_on_sc=False),
    scratch_types=[
        pltpu.VMEM_SHARED((NPAD,), jnp.float32),
        pltpu.VMEM((ROWS_PER_TILE_SC, EB), jnp.int32),
        pltpu.VMEM((ROWS_PER_TILE_SC, EB), jnp.float32),
        pltpu.VMEM((NODES_PER_TILE,), jnp.float32),
        pltpu.VMEM((NPAD,), jnp.float32),
        pltpu.VMEM((ROWS_PER_W, EB), jnp.int32),
        pltpu.VMEM((ROWS_PER_W, EB), jnp.int32),
        pltpu.VMEM((ROWS_PER_W, EB), jnp.float32),
        pltpu.VMEM((ROWS_PER_W, EB), jnp.float32),
        [pltpu.SemaphoreType.DMA for _ in range(NDS)],
    ],
)
def _edge_weights(row_hbm, col_hbm, wt_hbm, w_hbm,
                  deg_sp, colb, wtb, degb, disb, rwb, cwb, wwb, wout, dsems):
    c = lax.axis_index("c")
    s = lax.axis_index("s")
    wid = c * NS + s
    zero16 = jnp.zeros((L,), jnp.float32)

    def z_body(i, _):
        degb[pl.ds(i * L, L)] = zero16
        return 0
    lax.fori_loop(0, NODES_PER_TILE // L, z_body, 0)
    pltpu.sync_copy(degb, deg_sp.at[pl.ds(s * NODES_PER_TILE, NODES_PER_TILE)])
    plsc.subcore_barrier()

    pltpu.sync_copy(col_hbm.at[pl.ds(s * ROWS_PER_TILE_SC, ROWS_PER_TILE_SC)], colb)
    pltpu.sync_copy(wt_hbm.at[pl.ds(s * ROWS_PER_TILE_SC, ROWS_PER_TILE_SC)], wtb)

    def d_body(g, _):
        for k in range(NDS):
            j = g * NDS + k

            @pl.when(g >= 1)
            def _():
                pltpu.make_async_copy(wtb.at[0], deg_sp.at[colb.at[0]],
                                      dsems[k]).wait()
            pltpu.async_copy(wtb.at[j], deg_sp.at[colb.at[j]], dsems[k],
                             add=True)
        return 0
    lax.fori_loop(0, ROWS_PER_TILE_SC // NDS, d_body, 0)
    for k in range(NDS):
        pltpu.make_async_copy(wtb.at[0], deg_sp.at[colb.at[0]], dsems[k]).wait()
    plsc.subcore_barrier()

    pltpu.sync_copy(deg_sp.at[pl.ds(s * NODES_PER_TILE, NODES_PER_TILE)], degb)

    def r_body(i, _):
        x = degb[pl.ds(i * L, L)]
        yb = jnp.int32(0x5F3759DF) - (lax.bitcast_convert_type(x, jnp.int32) >> 1)
        y = lax.bitcast_convert_type(yb, jnp.float32)
        for _ in range(3):
            y = y * (1.5 - 0.5 * x * y * y)
        y = jnp.where(x == 0.0, 0.0, y)
        degb[pl.ds(i * L, L)] = y
        return 0
    lax.fori_loop(0, NODES_PER_TILE // L, r_body, 0)
    pltpu.sync_copy(degb, deg_sp.at[pl.ds(s * NODES_PER_TILE, NODES_PER_TILE)])
    plsc.subcore_barrier()

    pltpu.sync_copy(deg_sp, disb)
    base = wid * ROWS_PER_W
    pltpu.sync_copy(row_hbm.at[pl.ds(base, ROWS_PER_W)], rwb)
    pltpu.sync_copy(col_hbm.at[pl.ds(base, ROWS_PER_W)], cwb)
    pltpu.sync_copy(wt_hbm.at[pl.ds(base, ROWS_PER_W)], wwb)

    def w_body(j, _):
        for b in range(EB // L):
            rv = rwb[j, pl.ds(b * L, L)]
            cv = cwb[j, pl.ds(b * L, L)]
            wt = wwb[j, pl.ds(b * L, L)]
            dr = plsc.load_gather(disb, [rv])
            dc = plsc.load_gather(disb, [cv])
            wout[j, pl.ds(b * L, L)] = dr * wt * dc
        return 0
    lax.fori_loop(0, ROWS_PER_W, w_body, 0)
    pltpu.sync_copy(wout, w_hbm.at[pl.ds(base, ROWS_PER_W)])


@functools.partial(
    pl.kernel,
    out_type=jax.ShapeDtypeStruct((NC, NPAD, D), jnp.float32),
    mesh=_mesh,
    compiler_params=pltpu.CompilerParams(needs_layout_passes=False, use_tc_tiling_on_sc=False),
    scratch_types=[
        pltpu.VMEM_SHARED((NPAD, D), jnp.float32),
        pltpu.VMEM((NIB, 2, EB), jnp.int32),
        pltpu.VMEM((NIB, EB), jnp.float32),
        [pltpu.VMEM((EB, D), jnp.bfloat16) for _ in range(NBUF)],
        pltpu.VMEM((EB, D), jnp.float32),
        [pltpu.SemaphoreType.DMA for _ in range(NBUF)],
        [pltpu.SemaphoreType.DMA for _ in range(NIB)],
    ],
)
def _aggregate(h_hbm, rc_hbm, w_hbm, out_hbm,
               acc, rcb, wrb, bufs, mbuf, sems, isems):
    c = lax.axis_index("c")
    s = lax.axis_index("s")
    wid = c * NS + s
    zero16 = jnp.zeros((L,), jnp.float32)

    def z_body(i, _):
        for ch in range(D // L):
            mbuf[i, pl.ds(ch * L, L)] = zero16
        return 0
    lax.fori_loop(0, EB, z_body, 0)
    row0 = s * NODES_PER_TILE

    def zc_body(k, _):
        pltpu.sync_copy(mbuf, acc.at[pl.ds(row0 + k * EB, EB)])
        return 0
    lax.fori_loop(0, NODES_PER_TILE // EB, zc_body, 0)
    plsc.subcore_barrier()

    base = wid * ROWS_PER_W
    NB = ROWS_PER_W

    def _idx_start(j, slot):
        pltpu.async_copy(rc_hbm.at[base + j], rcb.at[slot], isems[slot])
        pltpu.async_copy(w_hbm.at[base + j], wrb.at[slot], isems[slot])

    def _idx_wait(slot):
        pltpu.make_async_copy(rc_hbm.at[base], rcb.at[slot], isems[slot]).wait()
        pltpu.make_async_copy(w_hbm.at[base], wrb.at[slot], isems[slot]).wait()

    for k in range(NIB):
        _idx_start(k, k)
    for k in range(NBUF):
        _idx_wait(k)
        pltpu.async_copy(h_hbm.at[rcb.at[k, 0]], bufs[k], sems[k])

    def g_body(g, _):
        for u in range(NIB):
            j = g * NIB + u
            b = u % NBUF
            pltpu.make_async_copy(h_hbm.at[rcb.at[u, 0]], bufs[b], sems[b]).wait()

            def s_body(grp, _):
                wv = wrb[u, pl.ds(grp * L, L)]
                for lane in range(L):
                    bw = _lane_bcast(wv, lane)
                    e = grp * L + lane
                    for ch in range(D // 32):
                        v = bufs[b][e, pl.ds(ch * 32, 32)]
                        lo, hi = plsc.unpack(v, format=plsc.PackFormat.INTERLEAVED)
                        mbuf[e, pl.ds(ch * 32, L)] = lo * bw
                        mbuf[e, pl.ds(ch * 32 + L, L)] = hi * bw
                return 0
            lax.fori_loop(0, EB // L, s_body, 0)

            @pl.when(j + NIB < NB)
            def _():
                _idx_start(j + NIB, u)

            @pl.when(j + NBUF < NB)
            def _():
                u2 = (u + NBUF) % NIB
                _idx_wait(u2)
                pltpu.async_copy(h_hbm.at[rcb.at[u2, 0]], bufs[b], sems[b])

            pltpu.sync_copy(mbuf, acc.at[rcb.at[u, 1]], add=True)
        return 0
    lax.fori_loop(0, NB // NIB, g_body, 0)
    plsc.subcore_barrier()

    pltpu.sync_copy(acc.at[pl.ds(row0, NODES_PER_TILE)],
                    out_hbm.at[c].at[pl.ds(row0, NODES_PER_TILE)])


def _shuf16(h):
    return h.reshape(N, D // 32, 2, 16).swapaxes(2, 3).reshape(N, D).astype(
        jnp.bfloat16)


def _bn(z, g, b):
    m = jnp.mean(z, axis=0, keepdims=True)
    v = jnp.mean(z * z, axis=0, keepdims=True) - m * m
    return (z - m) * lax.rsqrt(v + 1e-5) * g + b


def _dot(a, bt):
    return lax.dot_general(a, bt, (((1,), (0,)), ((), ())),
                           preferred_element_type=jnp.float32)


def _dense_body(h_ref, p_ref, w1t, b1, bng, bnb, w2t, b2, g, b, o_ref):
    out = h_ref[...] + p_ref[0, :N] + p_ref[1, :N]
    z = _dot(out, w1t[...]) + b1[...]
    hh = jnp.maximum(_bn(z, bng[...], bnb[...]), 0.0)
    t = _dot(hh, w2t[...]) + b2[...]
    o_ref[...] = jnp.maximum(_bn(t, g[...], b[...]), 0.0)


def _head_body(h_ref, p_ref, x_ref, w1t, b1, bng, bnb, w2t, b2, g, b,
               p0t, p0b, p2t, p2b, owt, ob, o_ref):
    out = h_ref[...] + p_ref[0, :N] + p_ref[1, :N]
    z = _dot(out, w1t[...]) + b1[...]
    hh = jnp.maximum(_bn(z, bng[...], bnb[...]), 0.0)
    t = _dot(hh, w2t[...]) + b2[...]
    h2 = jnp.maximum(_bn(t, g[...], b[...]), 0.0)
    oh = _dot(x_ref[...], p0t[...]) + p0b[...] + _dot(h2, p2t[...]) + p2b[...]
    oh = jnp.maximum(oh, 0.0)
    o_ref[...] = _dot(oh, owt[...]) + ob[...]


_dense_call = pl.pallas_call(
    _dense_body, out_shape=jax.ShapeDtypeStruct((N, D), jnp.float32))
_head_call = pl.pallas_call(
    _head_body, out_shape=jax.ShapeDtypeStruct((N, D), jnp.float32))


def kernel(x, A, weight, gpn0_W1, gpn0_b1, gpn0_bn_g, gpn0_bn_b, gpn0_W2,
           gpn0_b2, bn0_g, bn0_b, gpn1_W1, gpn1_b1, gpn1_bn_g, gpn1_bn_b,
           gpn1_W2, gpn1_b2, bn1_g, bn1_b, pred0_W, pred0_b, pred2_W, pred2_b,
           out_W, out_b):
    pad_idx = jnp.arange(E_PAD - E, dtype=jnp.int32) % N
    row3 = jnp.concatenate([A[0], pad_idx]).reshape(ROWS_ALL, EB)
    col3 = jnp.concatenate([A[1], pad_idx]).reshape(ROWS_ALL, EB)
    wt3 = jnp.concatenate(
        [weight, jnp.zeros(E_PAD - E, jnp.float32)]).reshape(ROWS_ALL, EB)
    r2 = lambda v: v.reshape(1, D)

    rc3 = jnp.stack([row3, col3], axis=1)
    w3 = _edge_weights(row3, col3, wt3)
    p0 = _aggregate(_shuf16(x), rc3, w3)
    h1 = _dense_call(x, p0, gpn0_W1.T, r2(gpn0_b1), r2(gpn0_bn_g),
                     r2(gpn0_bn_b), gpn0_W2.T, r2(gpn0_b2), r2(bn0_g),
                     r2(bn0_b))
    p1 = _aggregate(_shuf16(h1), rc3, w3)
    return _head_call(h1, p1, x, gpn1_W1.T, r2(gpn1_b1), r2(gpn1_bn_g),
                      r2(gpn1_bn_b), gpn1_W2.T, r2(gpn1_b2), r2(bn1_g),
                      r2(bn1_b), pred0_W.T, r2(pred0_b), pred2_W.T,
                      r2(pred2_b), out_W.T, r2(out_b))


# speedup vs baseline: 2.0853x; 2.0853x over previous
import functools

import jax
import jax.numpy as jnp
from jax import lax
from jax.experimental import pallas as pl
from jax.experimental.pallas import tpu as pltpu
from jax.experimental.pallas import tpu_sc as plsc

N = 10000
E = 320000
D = 128
NC = 2
NS = 16
NW = NC * NS
L = 16

EB = 128
ROWS_ALL = 2560
E_PAD = ROWS_ALL * EB
ROWS_PER_TILE_SC = ROWS_ALL // NS
ROWS_PER_W = ROWS_ALL // NW
NPAD = 10240
NODES_PER_TILE = NPAD // NS
NBUF = 2
NMB = 2
NIB = 2 * NBUF
NDS = 4

_mesh = plsc.VectorSubcoreMesh(
    core_axis_name="c", subcore_axis_name="s", num_cores=NC, num_subcores=NS)


def _lane_bcast(v, lane):
    idx = jnp.full((L, 1), lane, jnp.int32)
    dn = lax.GatherDimensionNumbers(
        offset_dims=(), collapsed_slice_dims=(0,), start_index_map=(0,))
    return lax.gather(v, idx, dn, slice_sizes=(1,),
                      mode=lax.GatherScatterMode.PROMISE_IN_BOUNDS)


@functools.partial(
    pl.kernel,
    out_type=jax.ShapeDtypeStruct((ROWS_ALL, EB), jnp.float32),
    mesh=_mesh,
    compiler_params=pltpu.CompilerParams(needs_layout_passes=False, use_tc_tiling_on_sc=False),
    scratch_types=[
        pltpu.VMEM_SHARED((NPAD,), jnp.float32),
        pltpu.VMEM((ROWS_PER_TILE_SC, EB), jnp.int32),
        pltpu.VMEM((ROWS_PER_TILE_SC, EB), jnp.float32),
        pltpu.VMEM((NODES_PER_TILE,), jnp.float32),
        pltpu.VMEM((NPAD,), jnp.float32),
        pltpu.VMEM((ROWS_PER_W, EB), jnp.int32),
        pltpu.VMEM((ROWS_PER_W, EB), jnp.int32),
        pltpu.VMEM((ROWS_PER_W, EB), jnp.float32),
        pltpu.VMEM((ROWS_PER_W, EB), jnp.float32),
        [pltpu.SemaphoreType.DMA for _ in range(NDS)],
    ],
)
def _edge_weights(row_hbm, col_hbm, wt_hbm, w_hbm,
                  deg_sp, colb, wtb, degb, disb, rwb, cwb, wwb, wout, dsems):
    c = lax.axis_index("c")
    s = lax.axis_index("s")
    wid = c * NS + s
    zero16 = jnp.zeros((L,), jnp.float32)

    def z_body(i, _):
        degb[pl.ds(i * L, L)] = zero16
        return 0
    lax.fori_loop(0, NODES_PER_TILE // L, z_body, 0)
    pltpu.sync_copy(degb, deg_sp.at[pl.ds(s * NODES_PER_TILE, NODES_PER_TILE)])
    plsc.subcore_barrier()

    pltpu.sync_copy(col_hbm.at[pl.ds(s * ROWS_PER_TILE_SC, ROWS_PER_TILE_SC)], colb)
    pltpu.sync_copy(wt_hbm.at[pl.ds(s * ROWS_PER_TILE_SC, ROWS_PER_TILE_SC)], wtb)

    def d_body(g, _):
        for k in range(NDS):
            j = g * NDS + k

            @pl.when(g >= 1)
            def _():
                pltpu.make_async_copy(wtb.at[0], deg_sp.at[colb.at[0]],
                                      dsems[k]).wait()
            pltpu.async_copy(wtb.at[j], deg_sp.at[colb.at[j]], dsems[k],
                             add=True)
        return 0
    lax.fori_loop(0, ROWS_PER_TILE_SC // NDS, d_body, 0)
    for k in range(NDS):
        pltpu.make_async_copy(wtb.at[0], deg_sp.at[colb.at[0]], dsems[k]).wait()
    plsc.subcore_barrier()

    pltpu.sync_copy(deg_sp.at[pl.ds(s * NODES_PER_TILE, NODES_PER_TILE)], degb)

    def r_body(i, _):
        x = degb[pl.ds(i * L, L)]
        yb = jnp.int32(0x5F3759DF) - (lax.bitcast_convert_type(x, jnp.int32) >> 1)
        y = lax.bitcast_convert_type(yb, jnp.float32)
        for _ in range(3):
            y = y * (1.5 - 0.5 * x * y * y)
        y = jnp.where(x == 0.0, 0.0, y)
        degb[pl.ds(i * L, L)] = y
        return 0
    lax.fori_loop(0, NODES_PER_TILE // L, r_body, 0)
    pltpu.sync_copy(degb, deg_sp.at[pl.ds(s * NODES_PER_TILE, NODES_PER_TILE)])
    plsc.subcore_barrier()

    pltpu.sync_copy(deg_sp, disb)
    base = wid * ROWS_PER_W
    pltpu.sync_copy(row_hbm.at[pl.ds(base, ROWS_PER_W)], rwb)
    pltpu.sync_copy(col_hbm.at[pl.ds(base, ROWS_PER_W)], cwb)
    pltpu.sync_copy(wt_hbm.at[pl.ds(base, ROWS_PER_W)], wwb)

    def w_body(j, _):
        for b in range(EB // L):
            rv = rwb[j, pl.ds(b * L, L)]
            cv = cwb[j, pl.ds(b * L, L)]
            wt = wwb[j, pl.ds(b * L, L)]
            dr = plsc.load_gather(disb, [rv])
            dc = plsc.load_gather(disb, [cv])
            wout[j, pl.ds(b * L, L)] = dr * wt * dc
        return 0
    lax.fori_loop(0, ROWS_PER_W, w_body, 0)
    pltpu.sync_copy(wout, w_hbm.at[pl.ds(base, ROWS_PER_W)])


@functools.partial(
    pl.kernel,
    out_type=jax.ShapeDtypeStruct((NC, NPAD, D), jnp.float32),
    mesh=_mesh,
    compiler_params=pltpu.CompilerParams(needs_layout_passes=False, use_tc_tiling_on_sc=False),
    scratch_types=[
        pltpu.VMEM_SHARED((NPAD, D), jnp.float32),
        pltpu.VMEM((NIB, 2, EB), jnp.int32),
        pltpu.VMEM((NIB, EB), jnp.float32),
        [pltpu.VMEM((EB, D), jnp.float32) for _ in range(NBUF)],
        [pltpu.SemaphoreType.DMA for _ in range(NBUF)],
        [pltpu.SemaphoreType.DMA for _ in range(NIB)],
    ],
)
def _aggregate(h_hbm, rc_hbm, w_hbm, out_hbm,
               acc, rcb, wrb, bufs, sems, isems):
    c = lax.axis_index("c")
    s = lax.axis_index("s")
    wid = c * NS + s
    zero16 = jnp.zeros((L,), jnp.float32)

    def z_body(i, _):
        for ch in range(D // L):
            bufs[0][i, pl.ds(ch * L, L)] = zero16
        return 0
    lax.fori_loop(0, EB, z_body, 0)
    row0 = s * NODES_PER_TILE

    def zc_body(k, _):
        pltpu.sync_copy(bufs[0], acc.at[pl.ds(row0 + k * EB, EB)])
        return 0
    lax.fori_loop(0, NODES_PER_TILE // EB, zc_body, 0)
    plsc.subcore_barrier()

    base = wid * ROWS_PER_W
    NB = ROWS_PER_W

    def _idx_start(j, slot):
        pltpu.async_copy(rc_hbm.at[base + j], rcb.at[slot], isems[slot])
        pltpu.async_copy(w_hbm.at[base + j], wrb.at[slot], isems[slot])

    def _idx_wait(slot):
        pltpu.make_async_copy(rc_hbm.at[base], rcb.at[slot], isems[slot]).wait()
        pltpu.make_async_copy(w_hbm.at[base], wrb.at[slot], isems[slot]).wait()

    for k in range(NIB):
        _idx_start(k, k)
    for k in range(NBUF):
        _idx_wait(k)
        pltpu.async_copy(h_hbm.at[rcb.at[k, 0]], bufs[k], sems[k])

    def g_body(g, _):
        for u in range(NIB):
            j = g * NIB + u
            b = u % NBUF
            pltpu.make_async_copy(h_hbm.at[rcb.at[u, 0]], bufs[b], sems[b]).wait()

            def s_body(grp, _):
                wv = wrb[u, pl.ds(grp * L, L)]
                for lane in range(L):
                    bw = _lane_bcast(wv, lane)
                    e = grp * L + lane
                    for ch in range(D // L):
                        sl = pl.ds(ch * L, L)
                        bufs[b][e, sl] = bufs[b][e, sl] * bw
                return 0
            lax.fori_loop(0, EB // L, s_body, 0)

            pltpu.sync_copy(bufs[b], acc.at[rcb.at[u, 1]], add=True)

            @pl.when(j + NIB < NB)
            def _():
                _idx_start(j + NIB, u)

            @pl.when(j + NBUF < NB)
            def _():
                u2 = (u + NBUF) % NIB
                _idx_wait(u2)
                pltpu.async_copy(h_hbm.at[rcb.at[u2, 0]], bufs[b], sems[b])
        return 0
    lax.fori_loop(0, NB // NIB, g_body, 0)
    plsc.subcore_barrier()

    pltpu.sync_copy(acc.at[pl.ds(row0, NODES_PER_TILE)],
                    out_hbm.at[c].at[pl.ds(row0, NODES_PER_TILE)])


def _bn(z, g, b):
    m = jnp.mean(z, axis=0, keepdims=True)
    v = jnp.mean(z * z, axis=0, keepdims=True) - m * m
    return (z - m) * lax.rsqrt(v + 1e-5) * g + b


def _dot(a, bt):
    return lax.dot_general(a, bt, (((1,), (0,)), ((), ())),
                           preferred_element_type=jnp.float32)


def _dense_body(h_ref, p_ref, w1t, b1, bng, bnb, w2t, b2, g, b, o_ref):
    out = h_ref[...] + p_ref[0, :N] + p_ref[1, :N]
    z = _dot(out, w1t[...]) + b1[...]
    hh = jnp.maximum(_bn(z, bng[...], bnb[...]), 0.0)
    t = _dot(hh, w2t[...]) + b2[...]
    o_ref[...] = jnp.maximum(_bn(t, g[...], b[...]), 0.0)


def _head_body(h_ref, p_ref, x_ref, w1t, b1, bng, bnb, w2t, b2, g, b,
               p0t, p0b, p2t, p2b, owt, ob, o_ref):
    out = h_ref[...] + p_ref[0, :N] + p_ref[1, :N]
    z = _dot(out, w1t[...]) + b1[...]
    hh = jnp.maximum(_bn(z, bng[...], bnb[...]), 0.0)
    t = _dot(hh, w2t[...]) + b2[...]
    h2 = jnp.maximum(_bn(t, g[...], b[...]), 0.0)
    oh = _dot(x_ref[...], p0t[...]) + p0b[...] + _dot(h2, p2t[...]) + p2b[...]
    oh = jnp.maximum(oh, 0.0)
    o_ref[...] = _dot(oh, owt[...]) + ob[...]


_dense_call = pl.pallas_call(
    _dense_body, out_shape=jax.ShapeDtypeStruct((N, D), jnp.float32))
_head_call = pl.pallas_call(
    _head_body, out_shape=jax.ShapeDtypeStruct((N, D), jnp.float32))


def kernel(x, A, weight, gpn0_W1, gpn0_b1, gpn0_bn_g, gpn0_bn_b, gpn0_W2,
           gpn0_b2, bn0_g, bn0_b, gpn1_W1, gpn1_b1, gpn1_bn_g, gpn1_bn_b,
           gpn1_W2, gpn1_b2, bn1_g, bn1_b, pred0_W, pred0_b, pred2_W, pred2_b,
           out_W, out_b):
    pad_idx = jnp.arange(E_PAD - E, dtype=jnp.int32) % N
    row3 = jnp.concatenate([A[0], pad_idx]).reshape(ROWS_ALL, EB)
    col3 = jnp.concatenate([A[1], pad_idx]).reshape(ROWS_ALL, EB)
    wt3 = jnp.concatenate(
        [weight, jnp.zeros(E_PAD - E, jnp.float32)]).reshape(ROWS_ALL, EB)
    r2 = lambda v: v.reshape(1, D)

    rc3 = jnp.stack([row3, col3], axis=1)
    w3 = _edge_weights(row3, col3, wt3)
    p0 = _aggregate(x, rc3, w3)
    h1 = _dense_call(x, p0, gpn0_W1.T, r2(gpn0_b1), r2(gpn0_bn_g),
                     r2(gpn0_bn_b), gpn0_W2.T, r2(gpn0_b2), r2(bn0_g),
                     r2(bn0_b))
    p1 = _aggregate(h1, rc3, w3)
    return _head_call(h1, p1, x, gpn1_W1.T, r2(gpn1_b1), r2(gpn1_bn_g),
                      r2(gpn1_bn_b), gpn1_W2.T, r2(gpn1_b2), r2(bn1_g),
                      r2(bn1_b), pred0_W.T, r2(pred0_b), pred2_W.T,
                      r2(pred2_b), out_W.T, r2(out_b))
